# baseline (speedup 1.0000x reference)
import functools

import jax
import jax.numpy as jnp
from jax.experimental import pallas as pl
from jax.experimental.pallas import tpu as pltpu


def _emb_kernel(x_hbm, aps_ref, w_ref, out_ref, xs, sem, *, row):
    copy = pltpu.make_async_copy(x_hbm.at[:, row, :], xs, sem)
    copy.start()
    copy.wait()
    logits = jax.lax.dot_general(
        xs[...],
        w_ref[...],
        dimension_numbers=(((1,), (1,)), ((), ())),
        preferred_element_type=jnp.float32,
    )
    mask = aps_ref[...] > 0
    out_ref[...] = jnp.where(mask, logits, jnp.zeros((), logits.dtype))


def kernel(input, agents_per_sample, W):
    B, A, S, D = input.shape
    V = W.shape[0]
    x3 = input.reshape(B, A * S, D)
    aps2 = agents_per_sample.reshape(B, 1)

    return pl.pallas_call(
        functools.partial(_emb_kernel, row=S - 1),
        out_shape=jax.ShapeDtypeStruct((B, V), input.dtype),
        grid=(1,),
        in_specs=[
            pl.BlockSpec(memory_space=pl.ANY),
            pl.BlockSpec((B, 1), lambda i: (0, 0)),
            pl.BlockSpec((V, D), lambda i: (0, 0)),
        ],
        out_specs=pl.BlockSpec((B, V), lambda i: (0, 0)),
        scratch_shapes=[
            pltpu.VMEM((B, D), jnp.float32),
            pltpu.SemaphoreType.DMA,
        ],
    )(x3, aps2, W)


# speedup vs baseline: 1.1829x; 1.1829x over previous
import functools

import jax
import jax.numpy as jnp
from jax.experimental import pallas as pl


def _emb_kernel(x_ref, aps_ref, w_ref, out_ref, *, row_off):
    B, R, D = x_ref.shape
    V = w_ref.shape[0]
    xflat = x_ref[...].reshape(B * R, D)
    logits_all = jax.lax.dot_general(
        xflat,
        w_ref[...],
        dimension_numbers=(((1,), (1,)), ((), ())),
        preferred_element_type=jnp.float32,
    ).reshape(B, R, V)
    rows = jax.lax.broadcasted_iota(jnp.int32, (B, R, V), 1)
    logits = jnp.sum(
        jnp.where(rows == row_off, logits_all, jnp.zeros((), logits_all.dtype)),
        axis=1,
    )
    mask = aps_ref[...] > 0
    out_ref[...] = jnp.where(mask, logits, jnp.zeros((), logits.dtype))


def kernel(input, agents_per_sample, W):
    B, A, S, D = input.shape
    V = W.shape[0]
    x3 = input.reshape(B, A * S, D)
    blk = (S - 1) // 8
    row_off = (S - 1) % 8
    aps2 = agents_per_sample.reshape(B, 1)

    return pl.pallas_call(
        functools.partial(_emb_kernel, row_off=row_off),
        out_shape=jax.ShapeDtypeStruct((B, V), input.dtype),
        grid=(1,),
        in_specs=[
            pl.BlockSpec((B, 8, D), lambda i: (0, blk, 0)),
            pl.BlockSpec((B, 1), lambda i: (0, 0)),
            pl.BlockSpec((V, D), lambda i: (0, 0)),
        ],
        out_specs=pl.BlockSpec((B, V), lambda i: (0, 0)),
    )(x3, aps2, W)
